# baseline (speedup 1.0000x reference)
import functools

import jax
import jax.numpy as jnp
from jax import lax
from jax.experimental import pallas as pl
from jax.experimental.pallas import tpu as pltpu
from jax.experimental.pallas import tpu_sc as plsc

_NUM_EMB = 8192
_EMB_DIM = 32
_COMM_COST = 1.0
_BETA = 0.25

_R = 256
_KCH = 2048
_N = 16384
_NB = _N // _R

_SC_NC = 2
_SC_NS = 16
_SC_NW = _SC_NC * _SC_NS
_B_PER_W = _N // _SC_NW
_IDX_CH = 128
_N_CH = _B_PER_W // _IDX_CH


def _tc_body(xb2_ref, emb_ref, row2_ref, e2_ref, idx_ref, loss_ref):
    xb2 = xb2_ref[...]
    row2 = row2_ref[...]
    best_dist = jnp.full((1, _R), jnp.inf, dtype=jnp.float32)
    best_idx = jnp.zeros((1, _R), dtype=jnp.int32)
    for j in range(_NUM_EMB // _KCH):
        embj = emb_ref[j * _KCH:(j + 1) * _KCH, :]
        e2j = e2_ref[j * _KCH:(j + 1) * _KCH, 0:1]
        mmT = lax.dot_general(
            embj, xb2, (((1,), (1,)), ((), ())),
            preferred_element_type=jnp.float32)
        d2T = (e2j + row2) - mmT
        distT = jnp.sqrt(jnp.maximum(d2T, 0.0))
        lmin = jnp.min(distT, axis=0, keepdims=True)
        iota = lax.broadcasted_iota(jnp.int32, (_KCH, _R), 0)
        lidx = jnp.min(jnp.where(distT == lmin, iota, _KCH),
                       axis=0, keepdims=True) + j * _KCH
        upd = lmin < best_dist
        best_idx = jnp.where(upd, lidx, best_idx)
        best_dist = jnp.where(upd, lmin, best_dist)
    idx_ref[0, :, :] = best_idx

    @pl.when(pl.program_id(0) == 0)
    def _():
        loss_ref[...] = jnp.zeros_like(loss_ref)

    loss_ref[...] += jnp.sum(best_dist * best_dist).reshape(1, 1)


def _tc_argmin(xb2, emb_weight, row2, e2):
    return pl.pallas_call(
        _tc_body,
        grid=(_NB,),
        in_specs=[
            pl.BlockSpec((_R, _EMB_DIM), lambda i: (i, 0)),
            pl.BlockSpec((_NUM_EMB, _EMB_DIM), lambda i: (0, 0)),
            pl.BlockSpec((1, _R), lambda i: (0, i)),
            pl.BlockSpec((_NUM_EMB, 1), lambda i: (0, 0)),
        ],
        out_specs=[
            pl.BlockSpec((1, 1, _R), lambda i: (i, 0, 0)),
            pl.BlockSpec((1, 1), lambda i: (0, 0)),
        ],
        out_shape=[
            jax.ShapeDtypeStruct((_NB, 1, _R), jnp.int32),
            jax.ShapeDtypeStruct((1, 1), jnp.float32),
        ],
    )(xb2, emb_weight, row2, e2)


@functools.cache
def _make_sc_gather():
    @functools.partial(
        pl.kernel,
        out_type=jax.ShapeDtypeStruct((_N, _EMB_DIM), jnp.float32),
        mesh=plsc.VectorSubcoreMesh(core_axis_name="c", subcore_axis_name="s"),
        compiler_params=pltpu.CompilerParams(use_tc_tiling_on_sc=False),
        scratch_types=[
            pltpu.VMEM((_N_CH, _IDX_CH), jnp.int32),
            pltpu.VMEM((_B_PER_W, _EMB_DIM), jnp.float32),
            pltpu.SemaphoreType.DMA,
        ],
    )
    def _sc_gather(table_hbm, idx_hbm, out_hbm, idx_v, rows_v, sem):
        wid = lax.axis_index("s") * _SC_NC + lax.axis_index("c")
        pltpu.sync_copy(idx_hbm.at[wid], idx_v)
        copies = [
            pltpu.async_copy(
                table_hbm.at[idx_v.at[c]],
                rows_v.at[pl.ds(c * _IDX_CH, _IDX_CH)],
                sem,
            )
            for c in range(_N_CH)
        ]
        for cp in copies:
            cp.wait()
        pltpu.sync_copy(rows_v, out_hbm.at[pl.ds(wid * _B_PER_W, _B_PER_W)])

    return _sc_gather


def kernel(x, emb_weight):
    xp = jnp.transpose(x, (0, 2, 3, 1))
    xf = xp.reshape(-1, _EMB_DIM)
    xb2 = (2.0 * xf).astype(jnp.bfloat16)
    row2 = jnp.sum(xf ** 2, axis=1)[None, :]
    e2 = jnp.sum(emb_weight ** 2, axis=1)[:, None]

    idx3, loss_sum = _tc_argmin(xb2, emb_weight, row2, e2)
    idx = idx3.reshape(_SC_NW, _N_CH, _IDX_CH)

    q = _make_sc_gather()(emb_weight, idx)

    c_loss = loss_sum[0, 0] * ((1.0 + _COMM_COST * _BETA) / (_N * _EMB_DIM))
    quantized = xf + (q - xf)
    quantized = quantized.reshape(xp.shape)
    quantized = jnp.transpose(quantized, (0, 3, 1, 2))
    return (c_loss, quantized)


# speedup vs baseline: 1.0038x; 1.0038x over previous
import functools

import jax
import jax.numpy as jnp
from jax import lax
from jax.experimental import pallas as pl
from jax.experimental.pallas import tpu as pltpu
from jax.experimental.pallas import tpu_sc as plsc

_NUM_EMB = 8192
_EMB_DIM = 32
_COMM_COST = 1.0
_BETA = 0.25

_R = 256
_KCH = 512
_N = 16384
_NB = _N // _R

_SC_NC = 2
_SC_NS = 16
_SC_NW = _SC_NC * _SC_NS
_B_PER_W = _N // _SC_NW
_IDX_CH = 128
_N_CH = _B_PER_W // _IDX_CH


def _tc_body(xb2_ref, emb_ref, row2_ref, e2_ref, idx_ref, loss_ref):
    xb2 = xb2_ref[...]
    row2 = row2_ref[...]
    best_dist = jnp.full((1, _R), jnp.inf, dtype=jnp.float32)
    best_idx = jnp.zeros((1, _R), dtype=jnp.int32)
    for j in range(_NUM_EMB // _KCH):
        embj = emb_ref[j * _KCH:(j + 1) * _KCH, :]
        e2j = e2_ref[j * _KCH:(j + 1) * _KCH, 0:1]
        mmT = lax.dot_general(
            embj, xb2, (((1,), (1,)), ((), ())),
            preferred_element_type=jnp.float32)
        d2T = (e2j + row2) - mmT
        distT = jnp.sqrt(jnp.maximum(d2T, 0.0))
        lmin = jnp.min(distT, axis=0, keepdims=True)
        iota = lax.broadcasted_iota(jnp.int32, (_KCH, _R), 0)
        lidx = jnp.min(jnp.where(distT == lmin, iota, _KCH),
                       axis=0, keepdims=True) + j * _KCH
        upd = lmin < best_dist
        best_idx = jnp.where(upd, lidx, best_idx)
        best_dist = jnp.where(upd, lmin, best_dist)
    idx_ref[0, :, :] = best_idx

    @pl.when(pl.program_id(0) == 0)
    def _():
        loss_ref[...] = jnp.zeros_like(loss_ref)

    loss_ref[...] += jnp.sum(best_dist * best_dist).reshape(1, 1)


def _tc_argmin(xb2, emb_weight, row2, e2):
    return pl.pallas_call(
        _tc_body,
        grid=(_NB,),
        in_specs=[
            pl.BlockSpec((_R, _EMB_DIM), lambda i: (i, 0)),
            pl.BlockSpec((_NUM_EMB, _EMB_DIM), lambda i: (0, 0)),
            pl.BlockSpec((1, _R), lambda i: (0, i)),
            pl.BlockSpec((_NUM_EMB, 1), lambda i: (0, 0)),
        ],
        out_specs=[
            pl.BlockSpec((1, 1, _R), lambda i: (i, 0, 0)),
            pl.BlockSpec((1, 1), lambda i: (0, 0)),
        ],
        out_shape=[
            jax.ShapeDtypeStruct((_NB, 1, _R), jnp.int32),
            jax.ShapeDtypeStruct((1, 1), jnp.float32),
        ],
    )(xb2, emb_weight, row2, e2)


@functools.cache
def _make_sc_gather():
    @functools.partial(
        pl.kernel,
        out_type=jax.ShapeDtypeStruct((_N, _EMB_DIM), jnp.float32),
        mesh=plsc.VectorSubcoreMesh(core_axis_name="c", subcore_axis_name="s"),
        compiler_params=pltpu.CompilerParams(use_tc_tiling_on_sc=False),
        scratch_types=[
            pltpu.VMEM((_N_CH, _IDX_CH), jnp.int32),
            pltpu.VMEM((_B_PER_W, _EMB_DIM), jnp.float32),
            pltpu.SemaphoreType.DMA,
        ],
    )
    def _sc_gather(table_hbm, idx_hbm, out_hbm, idx_v, rows_v, sem):
        wid = lax.axis_index("s") * _SC_NC + lax.axis_index("c")
        pltpu.sync_copy(idx_hbm.at[wid], idx_v)
        copies = [
            pltpu.async_copy(
                table_hbm.at[idx_v.at[c]],
                rows_v.at[pl.ds(c * _IDX_CH, _IDX_CH)],
                sem,
            )
            for c in range(_N_CH)
        ]
        for cp in copies:
            cp.wait()
        pltpu.sync_copy(rows_v, out_hbm.at[pl.ds(wid * _B_PER_W, _B_PER_W)])

    return _sc_gather


def kernel(x, emb_weight):
    xp = jnp.transpose(x, (0, 2, 3, 1))
    xf = xp.reshape(-1, _EMB_DIM)
    xb2 = (2.0 * xf).astype(jnp.bfloat16)
    row2 = jnp.sum(xf ** 2, axis=1)[None, :]
    e2 = jnp.sum(emb_weight ** 2, axis=1)[:, None]

    idx3, loss_sum = _tc_argmin(xb2, emb_weight, row2, e2)
    idx = idx3.reshape(_SC_NW, _N_CH, _IDX_CH)

    q = _make_sc_gather()(emb_weight, idx)

    c_loss = loss_sum[0, 0] * ((1.0 + _COMM_COST * _BETA) / (_N * _EMB_DIM))
    quantized = xf + (q - xf)
    quantized = quantized.reshape(xp.shape)
    quantized = jnp.transpose(quantized, (0, 3, 1, 2))
    return (c_loss, quantized)


# speedup vs baseline: 1.0130x; 1.0091x over previous
import functools

import jax
import jax.numpy as jnp
from jax import lax
from jax.experimental import pallas as pl
from jax.experimental.pallas import tpu as pltpu
from jax.experimental.pallas import tpu_sc as plsc

_NUM_EMB = 8192
_EMB_DIM = 32
_COMM_COST = 1.0
_BETA = 0.25

_R = 256
_KCH = 1024
_N = 16384
_NB = _N // _R

_SC_NC = 2
_SC_NS = 16
_SC_NW = _SC_NC * _SC_NS
_B_PER_W = _N // _SC_NW
_IDX_CH = 128
_N_CH = _B_PER_W // _IDX_CH


def _tc_body(xb2_ref, emb_ref, row2_ref, e2_ref, idx_ref, loss_ref):
    xb2 = xb2_ref[...]
    row2 = row2_ref[...]
    best_dist = jnp.full((1, _R), jnp.inf, dtype=jnp.float32)
    best_idx = jnp.zeros((1, _R), dtype=jnp.int32)
    for j in range(_NUM_EMB // _KCH):
        embj = emb_ref[j * _KCH:(j + 1) * _KCH, :]
        e2j = e2_ref[j * _KCH:(j + 1) * _KCH, 0:1]
        mmT = lax.dot_general(
            embj, xb2, (((1,), (1,)), ((), ())),
            preferred_element_type=jnp.float32)
        d2T = (e2j + row2) - mmT
        distT = jnp.sqrt(jnp.maximum(d2T, 0.0))
        lmin = jnp.min(distT, axis=0, keepdims=True)
        iota = lax.broadcasted_iota(jnp.int32, (_KCH, _R), 0)
        lidx = jnp.min(jnp.where(distT == lmin, iota, _KCH),
                       axis=0, keepdims=True) + j * _KCH
        upd = lmin < best_dist
        best_idx = jnp.where(upd, lidx, best_idx)
        best_dist = jnp.where(upd, lmin, best_dist)
    idx_ref[0, :, :] = best_idx

    @pl.when(pl.program_id(0) == 0)
    def _():
        loss_ref[...] = jnp.zeros_like(loss_ref)

    loss_ref[...] += jnp.sum(best_dist * best_dist).reshape(1, 1)


def _tc_argmin(xb2, emb_weight, row2, e2):
    return pl.pallas_call(
        _tc_body,
        grid=(_NB,),
        in_specs=[
            pl.BlockSpec((_R, _EMB_DIM), lambda i: (i, 0)),
            pl.BlockSpec((_NUM_EMB, _EMB_DIM), lambda i: (0, 0)),
            pl.BlockSpec((1, _R), lambda i: (0, i)),
            pl.BlockSpec((_NUM_EMB, 1), lambda i: (0, 0)),
        ],
        out_specs=[
            pl.BlockSpec((1, 1, _R), lambda i: (i, 0, 0)),
            pl.BlockSpec((1, 1), lambda i: (0, 0)),
        ],
        out_shape=[
            jax.ShapeDtypeStruct((_NB, 1, _R), jnp.int32),
            jax.ShapeDtypeStruct((1, 1), jnp.float32),
        ],
    )(xb2, emb_weight, row2, e2)


@functools.cache
def _make_sc_gather():
    @functools.partial(
        pl.kernel,
        out_type=jax.ShapeDtypeStruct((_N, _EMB_DIM), jnp.float32),
        mesh=plsc.VectorSubcoreMesh(core_axis_name="c", subcore_axis_name="s"),
        compiler_params=pltpu.CompilerParams(use_tc_tiling_on_sc=False),
        scratch_types=[
            pltpu.VMEM((_N_CH, _IDX_CH), jnp.int32),
            pltpu.VMEM((_B_PER_W, _EMB_DIM), jnp.float32),
            pltpu.SemaphoreType.DMA,
        ],
    )
    def _sc_gather(table_hbm, idx_hbm, out_hbm, idx_v, rows_v, sem):
        wid = lax.axis_index("s") * _SC_NC + lax.axis_index("c")
        pltpu.sync_copy(idx_hbm.at[wid], idx_v)
        copies = [
            pltpu.async_copy(
                table_hbm.at[idx_v.at[c]],
                rows_v.at[pl.ds(c * _IDX_CH, _IDX_CH)],
                sem,
            )
            for c in range(_N_CH)
        ]
        for cp in copies:
            cp.wait()
        pltpu.sync_copy(rows_v, out_hbm.at[pl.ds(wid * _B_PER_W, _B_PER_W)])

    return _sc_gather


def kernel(x, emb_weight):
    xp = jnp.transpose(x, (0, 2, 3, 1))
    xf = xp.reshape(-1, _EMB_DIM)
    xb2 = (2.0 * xf).astype(jnp.bfloat16)
    row2 = jnp.sum(xf ** 2, axis=1)[None, :]
    e2 = jnp.sum(emb_weight ** 2, axis=1)[:, None]

    idx3, loss_sum = _tc_argmin(xb2, emb_weight, row2, e2)
    idx = idx3.reshape(_SC_NW, _N_CH, _IDX_CH)

    q = _make_sc_gather()(emb_weight, idx)

    c_loss = loss_sum[0, 0] * ((1.0 + _COMM_COST * _BETA) / (_N * _EMB_DIM))
    quantized = xf + (q - xf)
    quantized = quantized.reshape(xp.shape)
    quantized = jnp.transpose(quantized, (0, 3, 1, 2))
    return (c_loss, quantized)
